# baseline (speedup 1.0000x reference)
import functools

import jax
import jax.numpy as jnp
from jax import lax
from jax.experimental import pallas as pl
from jax.experimental.pallas import tpu as pltpu
from jax.experimental.pallas import tpu_sc as plsc

EMBED_DIM = 64
NUM_ROWS = 16384
ROW_LEN = 50
NUM_CORES = 2
NUM_SUBCORES = 16
NUM_WORKERS = NUM_CORES * NUM_SUBCORES

TOTAL = NUM_ROWS * ROW_LEN
IBLK = 128
BLOCKS = ROW_LEN * (NUM_ROWS // IBLK)
BLOCKS_PER_WORKER = BLOCKS // NUM_WORKERS

_mesh = plsc.VectorSubcoreMesh(core_axis_name="c", subcore_axis_name="s")


@functools.partial(
    pl.kernel,
    mesh=_mesh,
    out_type=jax.ShapeDtypeStruct((TOTAL,), jnp.int32),
    scratch_types=[
        pltpu.VMEM((NUM_ROWS,), jnp.int32),
        pltpu.VMEM((NUM_ROWS,), jnp.int32),
    ],
)
def _detile_kernel(idxT_hbm, out_hbm, row_v0, row_v1):
    wid = lax.axis_index("s") * NUM_CORES + lax.axis_index("c")
    rows_v = [row_v0, row_v1]
    for rep in range(2):
        j = wid + NUM_WORKERS * rep
        @pl.when(j < ROW_LEN)
        def _():
            pltpu.sync_copy(idxT_hbm.at[j], rows_v[rep])
            pltpu.sync_copy(rows_v[rep],
                            out_hbm.at[pl.ds(j * NUM_ROWS, NUM_ROWS)])


@functools.partial(
    pl.kernel,
    mesh=_mesh,
    out_type=jax.ShapeDtypeStruct((ROW_LEN, EMBED_DIM, NUM_ROWS), jnp.float32),
    scratch_types=[
        pltpu.VMEM((2, IBLK), jnp.int32),
        pltpu.VMEM((2, IBLK, 2 * EMBED_DIM), jnp.float32),
        pltpu.VMEM((EMBED_DIM, IBLK), jnp.float32),
        pltpu.SemaphoreType.DMA,
        pltpu.SemaphoreType.DMA,
        pltpu.SemaphoreType.DMA,
        pltpu.SemaphoreType.DMA,
    ],
    compiler_params=pltpu.CompilerParams(needs_layout_passes=False),
)
def _gather_kernel(idx_hbm, table_hbm, out_hbm, idx_v, rows_v, tr_v,
                   si0, si1, sg0, sg1):
    sems_i = [si0, si1]
    sems_g = [sg0, sg1]
    wid = lax.axis_index("s") * NUM_CORES + lax.axis_index("c")
    blk0 = wid * BLOCKS_PER_WORKER
    lane = jnp.arange(16, dtype=jnp.int32)
    row_idx = [u * 16 + lane for u in range(IBLK // 16)]
    col_idx = [jnp.full((16,), k, jnp.int32) for k in range(EMBED_DIM)]

    def idx_off(t):
        blk = blk0 + t
        j = blk // (NUM_ROWS // IBLK)
        i0 = (blk % (NUM_ROWS // IBLK)) * IBLK
        return j, i0, j * NUM_ROWS + i0

    def start_idx(t, b):
        _, _, off = idx_off(t)
        pltpu.async_copy(idx_hbm.at[pl.ds(off, IBLK)], idx_v.at[b], sems_i[b])

    def wait_idx(t, b):
        _, _, off = idx_off(t)
        pltpu.make_async_copy(idx_hbm.at[pl.ds(off, IBLK)], idx_v.at[b],
                              sems_i[b]).wait()

    def start_gather(b):
        pltpu.async_copy(table_hbm.at[idx_v.at[b]], rows_v.at[b], sems_g[b])

    def wait_gather(b):
        pltpu.make_async_copy(table_hbm.at[idx_v.at[b]], rows_v.at[b],
                              sems_g[b]).wait()

    def transpose_and_store(t, b):
        j, i0, _ = idx_off(t)
        src = rows_v.at[b]
        for k in range(EMBED_DIM):
            trk = tr_v.at[k]
            for u in range(IBLK // 16):
                vals = plsc.load_gather(src, [row_idx[u], col_idx[k]])
                trk[pl.ds(u * 16, 16)] = vals
        pltpu.sync_copy(tr_v, out_hbm.at[j].at[:, pl.ds(i0, IBLK)])

    pltpu.sync_copy(idx_hbm.at[pl.ds(idx_off(0)[2], IBLK)], idx_v.at[0])
    start_gather(0)
    start_idx(1, 1)

    def step(t, b):
        nb = 1 - b
        wait_gather(b)
        wait_idx(t + 1, nb)
        start_gather(nb)
        start_idx(t + 2, b)
        transpose_and_store(t, b)

    def pair(p, _):
        t = p * 2
        step(t, 0)
        step(t + 1, 1)
        return 0

    lax.fori_loop(0, (BLOCKS_PER_WORKER - 2) // 2, pair, 0)

    t = BLOCKS_PER_WORKER - 2
    wait_gather(0)
    wait_idx(t + 1, 1)
    start_gather(1)
    transpose_and_store(t, 0)
    wait_gather(1)
    transpose_and_store(t + 1, 1)


def kernel(input, table):
    flat_idx = _detile_kernel(input.astype(jnp.int32).T)
    table_padded = jnp.pad(table, ((0, 0), (0, EMBED_DIM)))
    out = _gather_kernel(flat_idx, table_padded)
    return out.transpose(2, 0, 1)


# speedup vs baseline: 1.6057x; 1.6057x over previous
import functools

import jax
import jax.numpy as jnp
from jax import lax
from jax.experimental import pallas as pl
from jax.experimental.pallas import tpu as pltpu
from jax.experimental.pallas import tpu_sc as plsc

EMBED_DIM = 64
NUM_ROWS = 16384
ROW_LEN = 50
NUM_CORES = 2
NUM_SUBCORES = 16
NUM_WORKERS = NUM_CORES * NUM_SUBCORES

TOTAL = NUM_ROWS * ROW_LEN
PER_WORKER = TOTAL // NUM_WORKERS
ROWS_PER_WORKER = NUM_ROWS // NUM_WORKERS
ROWS_PER_CHUNK = 4
CHUNK = ROWS_PER_CHUNK * ROW_LEN
NUM_CHUNKS = PER_WORKER // CHUNK
NBUF = 4
LOOKAHEAD = NBUF - 1

_mesh = plsc.VectorSubcoreMesh(core_axis_name="c", subcore_axis_name="s")

I_BLOCK = NUM_ROWS // NUM_WORKERS
_TVECS = (I_BLOCK * ROW_LEN) // 16


@functools.partial(
    pl.kernel,
    mesh=_mesh,
    out_type=jax.ShapeDtypeStruct((TOTAL,), jnp.int32),
    scratch_types=[
        pltpu.VMEM((ROW_LEN, I_BLOCK), jnp.int32),
        pltpu.VMEM((I_BLOCK * ROW_LEN,), jnp.int32),
    ],
    compiler_params=pltpu.CompilerParams(needs_layout_passes=False),
)
def _detile_kernel(idxT_hbm, out_hbm, src_v, dst_v):
    wid = lax.axis_index("s") * NUM_CORES + lax.axis_index("c")
    i0 = wid * I_BLOCK
    pltpu.sync_copy(idxT_hbm.at[:, pl.ds(i0, I_BLOCK)], src_v)

    lane = jnp.arange(16, dtype=jnp.int32)

    def body(v, _):
        q = v * 16 + lane
        j = q % ROW_LEN
        il = q // ROW_LEN
        vals = plsc.load_gather(src_v, [j, il])
        dst_v[pl.ds(v * 16, 16)] = vals
        return 0

    lax.fori_loop(0, _TVECS, body, 0)
    pltpu.sync_copy(dst_v, out_hbm.at[pl.ds(wid * PER_WORKER, PER_WORKER)])


@functools.partial(
    pl.kernel,
    mesh=_mesh,
    out_type=jax.ShapeDtypeStruct((NUM_ROWS, ROW_LEN, EMBED_DIM), jnp.float32),
    scratch_types=[
        pltpu.VMEM((PER_WORKER,), jnp.int32),
        pltpu.VMEM((NBUF, CHUNK, EMBED_DIM), jnp.float32),
        pltpu.SemaphoreType.DMA,
        pltpu.SemaphoreType.DMA,
        pltpu.SemaphoreType.DMA,
        pltpu.SemaphoreType.DMA,
        pltpu.SemaphoreType.DMA,
        pltpu.SemaphoreType.DMA,
        pltpu.SemaphoreType.DMA,
        pltpu.SemaphoreType.DMA,
    ],
    compiler_params=pltpu.CompilerParams(use_tc_tiling_on_sc=False),
)
def _gather_kernel(idx_hbm, table_hbm, out_hbm, idx_v, rows_v,
                   sg0, sg1, sg2, sg3, sw0, sw1, sw2, sw3):
    sems_g = [sg0, sg1, sg2, sg3]
    sems_w = [sw0, sw1, sw2, sw3]
    wid = lax.axis_index("s") * NUM_CORES + lax.axis_index("c")
    base = wid * PER_WORKER
    base_row = wid * ROWS_PER_WORKER

    pltpu.sync_copy(idx_hbm.at[pl.ds(base, PER_WORKER)], idx_v)

    def start_gather(g, b):
        idx_slice = idx_v.at[pl.ds(pl.multiple_of(g * CHUNK, CHUNK), CHUNK)]
        pltpu.async_copy(table_hbm.at[idx_slice], rows_v.at[b], sems_g[b])

    def wait_gather(g, b):
        idx_slice = idx_v.at[pl.ds(pl.multiple_of(g * CHUNK, CHUNK), CHUNK)]
        pltpu.make_async_copy(table_hbm.at[idx_slice], rows_v.at[b],
                              sems_g[b]).wait()

    def start_wb(g, b):
        row0 = base_row + g * ROWS_PER_CHUNK
        for jr in range(ROWS_PER_CHUNK):
            src = rows_v.at[b].at[pl.ds(jr * ROW_LEN, ROW_LEN)]
            pltpu.async_copy(src, out_hbm.at[row0 + jr], sems_w[b])

    def wait_wb(g, b):
        row0 = base_row + g * ROWS_PER_CHUNK
        for jr in range(ROWS_PER_CHUNK):
            src = rows_v.at[b].at[pl.ds(jr * ROW_LEN, ROW_LEN)]
            pltpu.make_async_copy(src, out_hbm.at[row0 + jr],
                                  sems_w[b]).wait()

    for j in range(LOOKAHEAD):
        start_gather(j, j)

    start_gather(LOOKAHEAD, LOOKAHEAD % NBUF)
    wait_gather(0, 0)
    start_wb(0, 0)

    steady = NUM_CHUNKS - 1 - LOOKAHEAD
    groups = steady // NBUF

    def body(i, _):
        g0 = 1 + i * NBUF
        for db in range(NBUF):
            g = g0 + db
            b = (1 + db) % NBUF
            bj = (b + LOOKAHEAD) % NBUF
            wait_wb(g - 1, bj)
            start_gather(g + LOOKAHEAD, bj)
            wait_gather(g, b)
            start_wb(g, b)
        return 0

    lax.fori_loop(0, groups, body, 0)

    for g in range(NUM_CHUNKS - LOOKAHEAD, NUM_CHUNKS):
        b = g % NBUF
        wait_gather(g, b)
        start_wb(g, b)

    for g in range(NUM_CHUNKS - NBUF, NUM_CHUNKS):
        wait_wb(g, g % NBUF)


def kernel(input, table):
    flat_idx = _detile_kernel(input.astype(jnp.int32).T)
    return _gather_kernel(flat_idx, table)
